# baseline (speedup 1.0000x reference)
import functools

import jax
import jax.numpy as jnp
from jax import lax
from jax.experimental import pallas as pl
from jax.experimental.pallas import tpu as pltpu
from jax.experimental.pallas import tpu_sc as plsc

EMBED_DIM = 32
B_TOTAL = 4096 * 50

_info = plsc.get_sparse_core_info()
_NC, _NS = _info.num_cores, _info.num_subcores
NW = _NC * _NS
B_PER_W = B_TOTAL // NW
CHUNK = 800
N_CHUNKS = B_PER_W // CHUNK
NBUF = 2


def _make_kernel():
    mesh = plsc.VectorSubcoreMesh(core_axis_name="c", subcore_axis_name="s")

    @functools.partial(
        pl.kernel,
        mesh=mesh,
        out_type=jax.ShapeDtypeStruct((B_TOTAL, EMBED_DIM), jnp.float32),
        scratch_types=[
            pltpu.VMEM((B_PER_W,), jnp.int32),
            pltpu.VMEM((NBUF, CHUNK, EMBED_DIM), jnp.float32),
            pltpu.SemaphoreType.DMA,
            pltpu.SemaphoreType.DMA,
            pltpu.SemaphoreType.DMA,
            pltpu.SemaphoreType.DMA,
        ],
        compiler_params=pltpu.CompilerParams(use_tc_tiling_on_sc=False),
    )
    def gather_kernel(idx_hbm, table_hbm, out_hbm, idx_v, rows_v,
                      gsem0, gsem1, ssem0, ssem1):
        wid = lax.axis_index("s") * _NC + lax.axis_index("c")
        base = wid * B_PER_W
        gsems = (gsem0, gsem1)
        ssems = (ssem0, ssem1)

        pltpu.sync_copy(idx_hbm.at[pl.ds(base, B_PER_W)], idx_v)

        def gather_start(c, b):
            return pltpu.async_copy(
                table_hbm.at[idx_v.at[pl.ds(c * CHUNK, CHUNK)]],
                rows_v.at[b], gsems[b])

        def store_start(c, b):
            return pltpu.async_copy(
                rows_v.at[b], out_hbm.at[pl.ds(base + c * CHUNK, CHUNK)],
                ssems[b])

        g = [None] * NBUF
        s = [None] * NBUF
        g[0] = gather_start(0, 0)
        for c in range(N_CHUNKS):
            b = c % NBUF
            nb = (c + 1) % NBUF
            if c + 1 < N_CHUNKS:
                if s[nb] is not None:
                    s[nb].wait()
                g[nb] = gather_start(c + 1, nb)
            g[b].wait()
            s[b] = store_start(c, b)
        for b in range(NBUF):
            if s[b] is not None:
                s[b].wait()

    return gather_kernel


_gather = _make_kernel()


@jax.jit
def kernel(x, table):
    idx = x.reshape(-1).astype(jnp.int32)
    out = _gather(idx, table)
    return out.reshape(x.shape + (EMBED_DIM,))


# speedup vs baseline: 3.5153x; 3.5153x over previous
import functools

import jax
import jax.numpy as jnp
from jax import lax
from jax.experimental import pallas as pl
from jax.experimental.pallas import tpu as pltpu
from jax.experimental.pallas import tpu_sc as plsc

V = 1000000
D = 32
NJ = 50
NI = 4096
B_TOTAL = NJ * NI

_info = plsc.get_sparse_core_info()
NC, NS = _info.num_cores, _info.num_subcores
NW = NC * NS

CW = 768
CR = CW // 4
NFULL = V // CW
TAIL_V0 = NFULL * CW
TAIL_N = V - TAIL_V0
RM_ROWS = V // 4

W2 = 256
IPR = NI // W2
NUNITS = B_TOTAL // W2
UPT = NUNITS // NW


def _make_kernel():
    mesh = plsc.VectorSubcoreMesh(core_axis_name="c", subcore_axis_name="s")

    @functools.partial(
        pl.kernel,
        mesh=mesh,
        out_type=(
            jax.ShapeDtypeStruct((RM_ROWS, 128), jnp.float32),
            jax.ShapeDtypeStruct((NJ, D, NI), jnp.float32),
        ),
        scratch_types=[
            pltpu.VMEM((D, CW), jnp.float32),
            pltpu.VMEM((D, CW), jnp.float32),
            pltpu.VMEM((W2, 128), jnp.float32),
            pltpu.VMEM((W2, 128), jnp.float32),
            pltpu.VMEM((W2,), jnp.int32),
            pltpu.VMEM((W2,), jnp.int32),
            pltpu.VMEM((W2,), jnp.int32),
            pltpu.VMEM((W2,), jnp.int32),
            pltpu.VMEM((W2,), jnp.int32),
            pltpu.VMEM((W2,), jnp.int32),
            pltpu.SemaphoreType.DMA,
            pltpu.SemaphoreType.DMA,
            pltpu.SemaphoreType.DMA,
            pltpu.SemaphoreType.DMA,
            pltpu.SemaphoreType.DMA,
            pltpu.SemaphoreType.DMA,
            pltpu.SemaphoreType.DMA,
            pltpu.SemaphoreType.DMA,
            pltpu.SemaphoreType.DMA,
            pltpu.SemaphoreType.DMA,
            pltpu.SemaphoreType.REGULAR,
        ],
        compiler_params=pltpu.CompilerParams(
            use_tc_tiling_on_sc=True, needs_layout_passes=False),
    )
    def fused(xt, tt, tail_rm, rm, outn,
              C0, C1, G0, G1, IB0, IB1, GB0, GB1, M40, M41,
              lsem0, lsem1, ssem0, ssem1, isem0, isem1,
              gsem0, gsem1, osem0, osem1, bsem):
        sid = lax.axis_index("s")
        h = lax.axis_index("c")
        wid = sid * NC + h
        Cb = (C0, C1)
        Gb = (G0, G1)
        IBb = (IB0, IB1)
        GBb = (GB0, GB1)
        M4b = (M40, M41)
        lsem = (lsem0, lsem1)
        ssem = (ssem0, ssem1)
        isem = (isem0, isem1)
        gsem = (gsem0, gsem1)
        osem = (osem0, osem1)

        i16 = lax.iota(jnp.int32, 16)
        idiv4 = i16 >> 2
        lmod4 = i16 & 3
        sig = []
        colr = []
        for r in range(16):
            t = (i16 + r) & 15
            s = (t >> 2) + ((t & 3) << 2)
            sig.append(s)
            colr.append(lmod4 + (s << 2))

        def p1_transpose(C, G):
            def v0_body(vb, carry):
                v0 = vb * 16
                vvec = v0 + i16
                rv = (v0 >> 2) + idiv4
                for e0 in (0, 16):
                    vals = [plsc.load_gather(C, [sig[r] + e0, vvec])
                            for r in range(16)]
                    for r in range(16):
                        plsc.store_scatter(G, [rv, colr[r] + 4 * e0], vals[r])
                return carry
            lax.fori_loop(0, CW // 16, v0_body, 0, unroll=2)

        pltpu.async_copy(
            tt.at[:, pl.ds(pl.multiple_of(wid * CW, 256), CW)], C0, lsem0)

        def p1_body(g, carry):
            for b in (0, 1):
                k = 2 * g + b
                c = wid + 32 * k
                nxt = c + 32
                co = pl.multiple_of(c * CW, 256)
                no = pl.multiple_of(nxt * CW, 256)
                ro = pl.multiple_of(c * CR, 64)

                @pl.when(nxt < NFULL)
                def _():
                    pltpu.async_copy(
                        tt.at[:, pl.ds(no, CW)], Cb[1 - b], lsem[1 - b])

                @pl.when(c < NFULL)
                def _():
                    pltpu.make_async_copy(
                        tt.at[:, pl.ds(co, CW)], Cb[b], lsem[b]).wait()

                    @pl.when(k >= 2)
                    def _():
                        pltpu.make_async_copy(
                            Gb[b].at[pl.ds(0, CR), pl.ds(0, 128)],
                            rm.at[pl.ds(ro, CR), :], ssem[b]).wait()

                    p1_transpose(Cb[b], Gb[b])
                    pltpu.async_copy(
                        Gb[b].at[pl.ds(0, CR), pl.ds(0, 128)],
                        rm.at[pl.ds(ro, CR), :], ssem[b])
            return carry

        lax.fori_loop(0, 21, p1_body, 0)
        pltpu.make_async_copy(
            G0.at[pl.ds(0, CR), pl.ds(0, 128)],
            rm.at[pl.ds(0, CR), :], ssem0).wait()
        pltpu.make_async_copy(
            G1.at[pl.ds(0, CR), pl.ds(0, 128)],
            rm.at[pl.ds(0, CR), :], ssem1).wait()

        @pl.when(wid == NW - 1)
        def _():
            pltpu.sync_copy(tail_rm, G0.at[pl.ds(0, 16), pl.ds(0, 128)])
            pltpu.sync_copy(G0.at[pl.ds(0, 16), pl.ds(0, 128)],
                            rm.at[pl.ds(TAIL_V0 // 4, 16), :])

        plsc.subcore_barrier()

        @pl.when(sid == 0)
        def _():
            pl.semaphore_signal(bsem, 1, core_index=1 - h)
            pl.semaphore_wait(bsem, 1)

        plsc.subcore_barrier()

        base_u = wid * UPT

        def unit_jq(t):
            u = base_u + t
            return u // IPR, pl.multiple_of((u % IPR) * W2, W2)

        j0, q0 = unit_jq(0)
        pltpu.async_copy(xt.at[j0, pl.ds(q0, W2)], IB0, isem0)

        def p2_body(g, carry):
            for b in (0, 1):
                t = 2 * g + b

                @pl.when(t < UPT)
                def _():
                    j, q = unit_jq(t)
                    pltpu.make_async_copy(
                        xt.at[j, pl.ds(q, W2)], IBb[b], isem[b]).wait()

                    def gi(l0, cc):
                        iv = IBb[b][pl.ds(l0 * 16, 16)]
                        GBb[b][pl.ds(l0 * 16, 16)] = iv >> 2
                        M4b[b][pl.ds(l0 * 16, 16)] = iv & 3
                        return cc
                    lax.fori_loop(0, W2 // 16, gi, 0)

                    pltpu.async_copy(rm.at[GBb[b]], Gb[b], gsem[b])

                    @pl.when(t + 1 < UPT)
                    def _():
                        j2, q2 = unit_jq(t + 1)
                        pltpu.async_copy(
                            xt.at[j2, pl.ds(q2, W2)], IBb[1 - b], isem[1 - b])

                @pl.when((t >= 1) & (t <= UPT))
                def _():
                    tp = t - 1
                    bp = 1 - b
                    jp, qp = unit_jq(tp)
                    pltpu.make_async_copy(
                        rm.at[GBb[bp]], Gb[bp], gsem[bp]).wait()

                    @pl.when(tp >= 2)
                    def _():
                        pltpu.make_async_copy(
                            Cb[bp].at[:, pl.ds(0, W2)],
                            outn.at[jp, :, pl.ds(qp, W2)], osem[bp]).wait()

                    def l0_body(l0, cc):
                        p0 = l0 * 16
                        m4 = M4b[bp][pl.ds(p0, 16)]
                        pvec = i16 + p0
                        m4c = [m4 + (s << 2) for s in sig]
                        for e0 in (0, 16):
                            vs = [plsc.load_gather(
                                      Gb[bp], [pvec, m4c[r] + 4 * e0])
                                  for r in range(16)]
                            for r in range(16):
                                plsc.store_scatter(
                                    Cb[bp], [sig[r] + e0, pvec], vs[r])
                        return cc
                    lax.fori_loop(0, W2 // 16, l0_body, 0)

                    pltpu.async_copy(
                        Cb[bp].at[:, pl.ds(0, W2)],
                        outn.at[jp, :, pl.ds(qp, W2)], osem[bp])
            return carry

        lax.fori_loop(0, 13, p2_body, 0)
        pltpu.make_async_copy(
            C0.at[:, pl.ds(0, W2)], outn.at[0, :, pl.ds(0, W2)], osem0).wait()
        pltpu.make_async_copy(
            C1.at[:, pl.ds(0, W2)], outn.at[0, :, pl.ds(0, W2)], osem1).wait()

    return fused


_fused = _make_kernel()


@jax.jit
def kernel(x, table):
    xt = x.astype(jnp.int32).T
    tt = table.T
    tail_rm = (table[TAIL_V0:].reshape(16, 4, D)
               .transpose(0, 2, 1).reshape(16, 128))
    _, outn = _fused(xt, tt, tail_rm)
    return outn.transpose(2, 0, 1)


# speedup vs baseline: 3.8737x; 1.1020x over previous
import functools

import jax
import jax.numpy as jnp
from jax import lax
from jax.experimental import pallas as pl
from jax.experimental.pallas import tpu as pltpu
from jax.experimental.pallas import tpu_sc as plsc

V = 1000000
D = 32
NJ = 50
NI = 4096
B_TOTAL = NJ * NI

_info = plsc.get_sparse_core_info()
NC, NS = _info.num_cores, _info.num_subcores
NW = NC * NS

CW = 512
NFULL = V // CW
TAIL_V0 = NFULL * CW
TAIL_N = V - TAIL_V0
RM_ROWS = V // 4

W2 = 256
IPR = NI // W2
NUNITS = B_TOTAL // W2
UPT = NUNITS // NW


def _make_kernel():
    mesh = plsc.VectorSubcoreMesh(core_axis_name="c", subcore_axis_name="s")

    @functools.partial(
        pl.kernel,
        mesh=mesh,
        out_type=(
            jax.ShapeDtypeStruct((RM_ROWS, 128), jnp.float32),
            jax.ShapeDtypeStruct((NJ, D, NI), jnp.float32),
        ),
        scratch_types=[
            pltpu.VMEM((D, CW), jnp.float32),
            pltpu.VMEM((D, CW), jnp.float32),
            pltpu.VMEM((W2, 128), jnp.float32),
            pltpu.VMEM((W2, 128), jnp.float32),
            pltpu.VMEM((W2,), jnp.int32),
            pltpu.VMEM((W2,), jnp.int32),
            pltpu.VMEM((W2,), jnp.int32),
            pltpu.VMEM((W2,), jnp.int32),
            pltpu.VMEM((W2,), jnp.int32),
            pltpu.VMEM((W2,), jnp.int32),
            pltpu.SemaphoreType.DMA,
            pltpu.SemaphoreType.DMA,
            pltpu.SemaphoreType.DMA,
            pltpu.SemaphoreType.DMA,
            pltpu.SemaphoreType.DMA,
            pltpu.SemaphoreType.DMA,
            pltpu.SemaphoreType.DMA,
            pltpu.SemaphoreType.DMA,
            pltpu.SemaphoreType.DMA,
            pltpu.SemaphoreType.DMA,
            pltpu.SemaphoreType.REGULAR,
        ],
        compiler_params=pltpu.CompilerParams(
            use_tc_tiling_on_sc=True, needs_layout_passes=False),
    )
    def fused(xt, tt, tail_rm, rm, outn,
              C0, C1, G0, G1, IB0, IB1, GB0, GB1, M40, M41,
              lsem0, lsem1, ssem0, ssem1, isem0, isem1,
              gsem0, gsem1, osem0, osem1, bsem):
        sid = lax.axis_index("s")
        h = lax.axis_index("c")
        wid = sid * NC + h
        Cb = (C0, C1)
        Gb = (G0, G1)
        IBb = (IB0, IB1)
        GBb = (GB0, GB1)
        M4b = (M40, M41)
        lsem = (lsem0, lsem1)
        ssem = (ssem0, ssem1)
        isem = (isem0, isem1)
        gsem = (gsem0, gsem1)
        osem = (osem0, osem1)

        i16 = lax.iota(jnp.int32, 16)
        idiv4 = i16 >> 2
        lmod4 = i16 & 3
        sig = []
        colr = []
        for r in range(16):
            t = (i16 + r) & 15
            s = (t >> 2) + ((t & 3) << 2)
            sig.append(s)
            colr.append(lmod4 + (s << 2))

        def p1_transpose(C, G):
            def v0_body(vb, carry):
                v0 = vb * 16
                vvec = v0 + i16
                rv = (v0 >> 2) + idiv4
                for e0 in (0, 16):
                    vals = [plsc.load_gather(C, [sig[r] + e0, vvec])
                            for r in range(16)]
                    for r in range(16):
                        plsc.store_scatter(G, [rv, colr[r] + 4 * e0], vals[r])
                return carry
            lax.fori_loop(0, CW // 16, v0_body, 0, unroll=2)

        pltpu.async_copy(
            tt.at[:, pl.ds(pl.multiple_of(wid * CW, CW), CW)], C0, lsem0)

        def p1_body(g, carry):
            for b in (0, 1):
                k = 2 * g + b
                c = wid + 32 * k
                nxt = c + 32
                co = pl.multiple_of(c * CW, CW)
                no = pl.multiple_of(nxt * CW, CW)
                ro = pl.multiple_of(c * 128, 128)

                @pl.when(nxt < NFULL)
                def _():
                    pltpu.async_copy(
                        tt.at[:, pl.ds(no, CW)], Cb[1 - b], lsem[1 - b])

                @pl.when(c < NFULL)
                def _():
                    pltpu.make_async_copy(
                        tt.at[:, pl.ds(co, CW)], Cb[b], lsem[b]).wait()

                    @pl.when(k >= 2)
                    def _():
                        pltpu.make_async_copy(
                            Gb[b].at[pl.ds(0, 128), pl.ds(0, 128)],
                            rm.at[pl.ds(ro, 128), :], ssem[b]).wait()

                    p1_transpose(Cb[b], Gb[b])
                    pltpu.async_copy(
                        Gb[b].at[pl.ds(0, 128), pl.ds(0, 128)],
                        rm.at[pl.ds(ro, 128), :], ssem[b])
            return carry

        lax.fori_loop(0, 31, p1_body, 0)
        pltpu.make_async_copy(
            G0.at[pl.ds(0, 128), pl.ds(0, 128)],
            rm.at[pl.ds(0, 128), :], ssem0).wait()
        pltpu.make_async_copy(
            G1.at[pl.ds(0, 128), pl.ds(0, 128)],
            rm.at[pl.ds(0, 128), :], ssem1).wait()

        @pl.when(wid == NW - 1)
        def _():
            pltpu.sync_copy(tail_rm, G0.at[pl.ds(0, 16), pl.ds(0, 128)])
            pltpu.sync_copy(G0.at[pl.ds(0, 16), pl.ds(0, 128)],
                            rm.at[pl.ds(TAIL_V0 // 4, 16), :])

        plsc.subcore_barrier()

        @pl.when(sid == 0)
        def _():
            pl.semaphore_signal(bsem, 1, core_index=1 - h)
            pl.semaphore_wait(bsem, 1)

        plsc.subcore_barrier()

        base_u = wid * UPT

        def unit_jq(t):
            u = base_u + t
            return u // IPR, pl.multiple_of((u % IPR) * W2, W2)

        j0, q0 = unit_jq(0)
        pltpu.async_copy(xt.at[j0, pl.ds(q0, W2)], IB0, isem0)

        def p2_body(g, carry):
            for b in (0, 1):
                t = 2 * g + b

                @pl.when(t < UPT)
                def _():
                    j, q = unit_jq(t)
                    pltpu.make_async_copy(
                        xt.at[j, pl.ds(q, W2)], IBb[b], isem[b]).wait()

                    def gi(l0, cc):
                        iv = IBb[b][pl.ds(l0 * 16, 16)]
                        GBb[b][pl.ds(l0 * 16, 16)] = iv >> 2
                        M4b[b][pl.ds(l0 * 16, 16)] = iv & 3
                        return cc
                    lax.fori_loop(0, W2 // 16, gi, 0)

                    pltpu.async_copy(rm.at[GBb[b]], Gb[b], gsem[b])

                    @pl.when(t + 1 < UPT)
                    def _():
                        j2, q2 = unit_jq(t + 1)
                        pltpu.async_copy(
                            xt.at[j2, pl.ds(q2, W2)], IBb[1 - b], isem[1 - b])

                @pl.when((t >= 1) & (t <= UPT))
                def _():
                    tp = t - 1
                    bp = 1 - b
                    jp, qp = unit_jq(tp)
                    pltpu.make_async_copy(
                        rm.at[GBb[bp]], Gb[bp], gsem[bp]).wait()

                    @pl.when(tp >= 2)
                    def _():
                        pltpu.make_async_copy(
                            Cb[bp].at[:, pl.ds(0, W2)],
                            outn.at[jp, :, pl.ds(qp, W2)], osem[bp]).wait()

                    def l0_body(l0, cc):
                        p0 = l0 * 16
                        m4 = M4b[bp][pl.ds(p0, 16)]
                        pvec = i16 + p0
                        m4c = [m4 + (s << 2) for s in sig]
                        for e0 in (0, 16):
                            vs = [plsc.load_gather(
                                      Gb[bp], [pvec, m4c[r] + 4 * e0])
                                  for r in range(16)]
                            for r in range(16):
                                plsc.store_scatter(
                                    Cb[bp], [sig[r] + e0, pvec], vs[r])
                        return cc
                    lax.fori_loop(0, W2 // 16, l0_body, 0)

                    pltpu.async_copy(
                        Cb[bp].at[:, pl.ds(0, W2)],
                        outn.at[jp, :, pl.ds(qp, W2)], osem[bp])
            return carry

        lax.fori_loop(0, 13, p2_body, 0)
        pltpu.make_async_copy(
            C0.at[:, pl.ds(0, W2)], outn.at[0, :, pl.ds(0, W2)], osem0).wait()
        pltpu.make_async_copy(
            C1.at[:, pl.ds(0, W2)], outn.at[0, :, pl.ds(0, W2)], osem1).wait()

    return fused


_fused = _make_kernel()


@jax.jit
def kernel(x, table):
    xt = x.astype(jnp.int32).T
    tt = table.T
    tail_rm = (table[TAIL_V0:].reshape(16, 4, D)
               .transpose(0, 2, 1).reshape(16, 128))
    _, outn = _fused(xt, tt, tail_rm)
    return outn.transpose(2, 0, 1)
